# Initial kernel scaffold; baseline (speedup 1.0000x reference)
import jax
import jax.numpy as jnp
from jax.experimental import pallas as pl


def kernel(W, edge_index, eps_0, W1_0, b1_0, g_0, bt_0, W2_0, b2_0, eps_1, W1_1, b1_1, g_1, bt_1, W2_1, b2_1, eps_2, W1_2, b1_2, g_2, bt_2, W2_2, b2_2):
    raise NotImplementedError("write your pallas kernel here")


# speedup vs baseline: 6.7948x; 6.7948x over previous
import jax
import jax.numpy as jnp
from jax.experimental import pallas as pl
from jax.experimental.pallas import tpu as pltpu

_N = 512
_E = 4096
_CH = 16
_NN = _N * _N
_L = 128
_G = _L // _CH
_PR = _NN * _CH // _L

_GP = 4
_BC = (_N * _CH) // _GP
_GF = 8
_BR = _PR // _GF
_BN = _N // _GF


def _sel_pack(rows):
    return (jax.lax.broadcasted_iota(jnp.int32, (rows, _CH), 0) % _CH
            == jax.lax.broadcasted_iota(jnp.int32, (rows, _CH), 1)
            ).astype(jnp.float32)


def _sel_unpack():
    return (jax.lax.broadcasted_iota(jnp.int32, (_CH, _L), 0)
            == jax.lax.broadcasted_iota(jnp.int32, (_CH, _L), 1) % _CH
            ).astype(jnp.float32)


def _mm_body(x_ref, w_ref, o_ref):
    o_ref[...] = jnp.dot(x_ref[...], w_ref[...],
                         preferred_element_type=jnp.float32)


def _adj_body(ei_ref, a_ref):
    src = ei_ref[0, :]
    dst = ei_ref[1, :]
    rows = jax.lax.broadcasted_iota(jnp.int32, (_N, _E), 0)
    u = (rows == dst[None, :]).astype(jnp.float32)
    cols = jax.lax.broadcasted_iota(jnp.int32, (_E, _N), 1)
    v = (cols == src[:, None]).astype(jnp.float32)
    a_ref[...] = jnp.dot(u, v, preferred_element_type=jnp.float32)


def _prop_body(epsp_ref, a_ref, y_ref, h_ref, s1_ref, s2_ref):
    k = epsp_ref[0]
    y = y_ref[...]
    hh = jnp.dot(a_ref[...], y, preferred_element_type=jnp.float32) + k * y
    h_ref[...] = hh
    p = _sel_pack(_BC)
    s1 = jnp.dot(jnp.sum(hh, axis=0, keepdims=True), p,
                 preferred_element_type=jnp.float32)
    s2 = jnp.dot(jnp.sum(hh * hh, axis=0, keepdims=True), p,
                 preferred_element_type=jnp.float32)
    s1_ref[...] = s1[None]
    s2_ref[...] = s2[None]


def _norm_coefs(s1_ref, s2_ref, vecs_ref):
    s1 = jnp.sum(s1_ref[...], axis=0)
    s2 = jnp.sum(s2_ref[...], axis=0)
    mu = s1 / _NN
    var = s2 / _NN - mu * mu
    g = vecs_ref[0:1, :]
    bt = vecs_ref[1:2, :]
    a = jax.lax.rsqrt(var + 1e-5) * g
    d = bt - mu * a
    u = _sel_unpack()
    return jnp.dot(a, u), jnp.dot(d, u)


def _finish_body(s1_ref, s2_ref, vecs_ref, c_ref, bias_ref, h_ref, o_ref):
    a, d = _norm_coefs(s1_ref, s2_ref, vecs_ref)
    x = jnp.maximum(h_ref[...] * a + d, 0.0)
    o_ref[...] = (jnp.dot(x, c_ref[...], preferred_element_type=jnp.float32)
                  + bias_ref[0:1, :])


def _final_body(s1_ref, s2_ref, vecs_ref, w2_ref, h_ref, o_ref):
    a, d = _norm_coefs(s1_ref, s2_ref, vecs_ref)
    x = jnp.maximum(h_ref[...] * a + d, 0.0)
    rows_per_node = _N * _CH // _L
    r = jnp.sum(x.reshape(_BN, rows_per_node, _L), axis=1)
    r16 = jnp.dot(r, _sel_pack(_L), preferred_element_type=jnp.float32)
    o_ref[...] = (jnp.dot(r16, w2_ref[...], preferred_element_type=jnp.float32)
                  + _N * vecs_ref[2:3, :])


def _channel_mm(x, w128):
    return pl.pallas_call(
        _mm_body,
        grid=(_GF,),
        in_specs=[pl.BlockSpec((_BR, _L), lambda i: (i, 0)),
                  pl.BlockSpec((_L, _L), lambda i: (0, 0))],
        out_specs=pl.BlockSpec((_BR, _L), lambda i: (i, 0)),
        out_shape=jax.ShapeDtypeStruct((_PR, _L), jnp.float32),
    )(x, w128)


def _adjacency(edge_index):
    return pl.pallas_call(
        _adj_body,
        in_specs=[pl.BlockSpec((2, _E), lambda: (0, 0))],
        out_specs=pl.BlockSpec((_N, _N), lambda: (0, 0)),
        out_shape=jax.ShapeDtypeStruct((_N, _N), jnp.float32),
    )(edge_index)


def _propagate(epsp, adj, y):
    y2d = y.reshape(_N, _N * _CH)
    return pl.pallas_call(
        _prop_body,
        grid=(_GP,),
        in_specs=[pl.BlockSpec(memory_space=pltpu.SMEM),
                  pl.BlockSpec((_N, _N), lambda j: (0, 0)),
                  pl.BlockSpec((_N, _BC), lambda j: (0, j))],
        out_specs=(pl.BlockSpec((_N, _BC), lambda j: (0, j)),
                   pl.BlockSpec((1, 1, _CH), lambda j: (j, 0, 0)),
                   pl.BlockSpec((1, 1, _CH), lambda j: (j, 0, 0))),
        out_shape=(jax.ShapeDtypeStruct((_N, _N * _CH), jnp.float32),
                   jax.ShapeDtypeStruct((_GP, 1, _CH), jnp.float32),
                   jax.ShapeDtypeStruct((_GP, 1, _CH), jnp.float32)),
    )(epsp, adj, y2d)


def _finish(s1, s2, vecs, c128, bias, h2d):
    h = h2d.reshape(_PR, _L)
    return pl.pallas_call(
        _finish_body,
        grid=(_GF,),
        in_specs=[pl.BlockSpec((_GP, 1, _CH), lambda i: (0, 0, 0)),
                  pl.BlockSpec((_GP, 1, _CH), lambda i: (0, 0, 0)),
                  pl.BlockSpec((8, _CH), lambda i: (0, 0)),
                  pl.BlockSpec((_L, _L), lambda i: (0, 0)),
                  pl.BlockSpec((1, _L), lambda i: (0, 0)),
                  pl.BlockSpec((_BR, _L), lambda i: (i, 0))],
        out_specs=pl.BlockSpec((_BR, _L), lambda i: (i, 0)),
        out_shape=jax.ShapeDtypeStruct((_PR, _L), jnp.float32),
    )(s1, s2, vecs, c128, bias, h)


def _final(s1, s2, vecs, w2, h2d):
    h = h2d.reshape(_PR, _L)
    return pl.pallas_call(
        _final_body,
        grid=(_GF,),
        in_specs=[pl.BlockSpec((_GP, 1, _CH), lambda i: (0, 0, 0)),
                  pl.BlockSpec((_GP, 1, _CH), lambda i: (0, 0, 0)),
                  pl.BlockSpec((8, _CH), lambda i: (0, 0)),
                  pl.BlockSpec((_CH, _CH), lambda i: (0, 0)),
                  pl.BlockSpec((_BR, _L), lambda i: (i, 0))],
        out_specs=pl.BlockSpec((_BN, _CH), lambda i: (i, 0)),
        out_shape=jax.ShapeDtypeStruct((_N, _CH), jnp.float32),
    )(s1, s2, vecs, w2, h)


def kernel(W, edge_index, eps_0, W1_0, b1_0, g_0, bt_0, W2_0, b2_0,
           eps_1, W1_1, b1_1, g_1, bt_1, W2_1, b2_1,
           eps_2, W1_2, b1_2, g_2, bt_2, W2_2, b2_2):
    eye8 = jnp.eye(_G, dtype=jnp.float32)
    d1_0 = jnp.kron(eye8, W1_0)
    c0 = jnp.kron(eye8, W2_0 @ W1_1)
    c1 = jnp.kron(eye8, W2_1 @ W1_2)
    bias0 = jnp.tile(b2_0 @ W1_1, _G)[None]
    bias1 = jnp.tile(b2_1 @ W1_2, _G)[None]
    pad = jnp.zeros((5, _CH), jnp.float32)
    vecs0 = jnp.concatenate([g_0[None], bt_0[None], pad, pad[:1]], axis=0)
    vecs1 = jnp.concatenate([g_1[None], bt_1[None], pad, pad[:1]], axis=0)
    vecs2 = jnp.concatenate([g_2[None], bt_2[None], b2_2[None], pad], axis=0)
    epsp = [1.0 + eps_0, 1.0 + eps_1, 1.0 + eps_2]

    adj = _adjacency(edge_index)
    y = _channel_mm(W.reshape(_PR, _L), d1_0)

    h, s1, s2 = _propagate(epsp[0], adj, y)
    y = _finish(s1, s2, vecs0, c0, bias0, h)
    h, s1, s2 = _propagate(epsp[1], adj, y)
    y = _finish(s1, s2, vecs1, c1, bias1, h)
    h, s1, s2 = _propagate(epsp[2], adj, y)
    return _final(s1, s2, vecs2, W2_2, h)


# speedup vs baseline: 17.9318x; 2.6391x over previous
import jax
import jax.numpy as jnp
from jax.experimental import pallas as pl
from jax.experimental.pallas import tpu as pltpu

_N = 512
_E = 4096
_CH = 16
_NN = _N * _N
_LN = _N * _CH
_ECHUNK = 1024
_JB = 2048
_NJ = _LN // _JB
_KB = 128
_NK = _LN // _KB


def _sel(rows, cols, mod):
    return (jax.lax.broadcasted_iota(jnp.int32, (rows, cols), 0) % mod
            == jax.lax.broadcasted_iota(jnp.int32, (rows, cols), 1) % mod
            ).astype(jnp.float32)


def _body(epsp_ref, ei_ref, d1_ref, c0_ref, c1_ref, w2_ref, vecs_ref,
          w_hbm, o_ref, a_ref, y_ref, h_ref, r_ref, sem):
    cp = pltpu.make_async_copy(w_hbm, h_ref, sem)
    cp.start()

    for t in range(_E // _ECHUNK):
        src = ei_ref[0, pl.ds(t * _ECHUNK, _ECHUNK)]
        dst = ei_ref[1, pl.ds(t * _ECHUNK, _ECHUNK)]
        rows = jax.lax.broadcasted_iota(jnp.int32, (_N, _ECHUNK), 0)
        u = (rows == dst[None, :]).astype(jnp.float32)
        cols = jax.lax.broadcasted_iota(jnp.int32, (_ECHUNK, _N), 1)
        v = (cols == src[:, None]).astype(jnp.float32)
        uv = jnp.dot(u, v, preferred_element_type=jnp.float32)
        if t == 0:
            a_ref[...] = uv
        else:
            a_ref[...] += uv

    cp.wait()

    def y0_step(k, _):
        js = pl.ds(k * _KB, _KB)
        y_ref[:, js] = jnp.dot(h_ref[:, js], d1_ref[...],
                               preferred_element_type=jnp.float32)
        return 0

    jax.lax.fori_loop(0, _NK, y0_step, 0)

    for layer in range(3):
        k_eps = epsp_ref[layer]

        def prop_step(j, carry):
            s1, s2 = carry
            js = pl.ds(j * _JB, _JB)
            yb = y_ref[:, js]
            hb = jnp.dot(a_ref[...], yb,
                         preferred_element_type=jnp.float32) + k_eps * yb
            h_ref[:, js] = hb
            p = _sel(_JB, _CH, _CH)
            s1 = s1 + jnp.dot(jnp.sum(hb, axis=0, keepdims=True), p,
                              preferred_element_type=jnp.float32)
            s2 = s2 + jnp.dot(jnp.sum(hb * hb, axis=0, keepdims=True), p,
                              preferred_element_type=jnp.float32)
            return s1, s2

        zero16 = jnp.zeros((1, _CH), jnp.float32)
        s1, s2 = jax.lax.fori_loop(0, _NJ, prop_step, (zero16, zero16))

        mu = s1 / _NN
        var = s2 / _NN - mu * mu
        g = vecs_ref[layer, 0:1, :]
        bt = vecs_ref[layer, 1:2, :]
        av = jax.lax.rsqrt(var + 1e-5) * g
        dv = bt - mu * av
        u = _sel(_CH, _KB, _CH)
        a128 = jnp.dot(av, u, preferred_element_type=jnp.float32)
        d128 = jnp.dot(dv, u, preferred_element_type=jnp.float32)

        if layer < 2:
            c_ref = c0_ref if layer == 0 else c1_ref
            bias = vecs_ref[layer, 2:3, :]
            b128 = jnp.dot(bias, u, preferred_element_type=jnp.float32)

            def fin_step(k, _):
                js = pl.ds(k * _KB, _KB)
                x = jnp.maximum(h_ref[:, js] * a128 + d128, 0.0)
                y_ref[:, js] = jnp.dot(
                    x, c_ref[...], preferred_element_type=jnp.float32) + b128
                return 0

            jax.lax.fori_loop(0, _NK, fin_step, 0)
        else:
            pf = _sel(_KB, _CH, _CH)

            def red_step(k, _):
                js = pl.ds(k * _KB, _KB)
                x = jnp.maximum(h_ref[:, js] * a128 + d128, 0.0)
                rk = jnp.dot(x, pf, preferred_element_type=jnp.float32)
                r_ref[...] = jnp.where(k == 0, rk, r_ref[...] + rk)
                return 0

            jax.lax.fori_loop(0, _NK, red_step, 0)

            b2 = vecs_ref[layer, 2:3, :]
            o_ref[...] = (jnp.dot(r_ref[...], w2_ref[...],
                                  preferred_element_type=jnp.float32)
                          + _N * b2)


def kernel(W, edge_index, eps_0, W1_0, b1_0, g_0, bt_0, W2_0, b2_0,
           eps_1, W1_1, b1_1, g_1, bt_1, W2_1, b2_1,
           eps_2, W1_2, b1_2, g_2, bt_2, W2_2, b2_2):
    eye8 = jnp.eye(_KB // _CH, dtype=jnp.float32)
    d1 = jnp.kron(eye8, W1_0)
    c0 = jnp.kron(eye8, W2_0 @ W1_1)
    c1 = jnp.kron(eye8, W2_1 @ W1_2)
    pad = jnp.zeros((5, _CH), jnp.float32)
    vecs = jnp.stack([
        jnp.concatenate([g_0[None], bt_0[None], (b2_0 @ W1_1)[None], pad]),
        jnp.concatenate([g_1[None], bt_1[None], (b2_1 @ W1_2)[None], pad]),
        jnp.concatenate([g_2[None], bt_2[None], b2_2[None], pad]),
    ])
    epsp = 1.0 + jnp.concatenate([eps_0, eps_1, eps_2])

    return pl.pallas_call(
        _body,
        in_specs=[pl.BlockSpec(memory_space=pltpu.SMEM),
                  pl.BlockSpec(memory_space=pltpu.VMEM),
                  pl.BlockSpec(memory_space=pltpu.VMEM),
                  pl.BlockSpec(memory_space=pltpu.VMEM),
                  pl.BlockSpec(memory_space=pltpu.VMEM),
                  pl.BlockSpec(memory_space=pltpu.VMEM),
                  pl.BlockSpec(memory_space=pltpu.VMEM),
                  pl.BlockSpec(memory_space=pl.ANY)],
        out_specs=pl.BlockSpec(memory_space=pltpu.VMEM),
        out_shape=jax.ShapeDtypeStruct((_N, _CH), jnp.float32),
        scratch_shapes=[pltpu.VMEM((_N, _N), jnp.float32),
                        pltpu.VMEM((_N, _LN), jnp.float32),
                        pltpu.VMEM((_N, _LN), jnp.float32),
                        pltpu.VMEM((_N, _CH), jnp.float32),
                        pltpu.SemaphoreType.DMA],
    )(epsp, edge_index, d1, c0, c1, W2_2, vecs, W.reshape(_N, _LN))


# speedup vs baseline: 18.2142x; 1.0157x over previous
import jax
import jax.numpy as jnp
from jax.experimental import pallas as pl
from jax.experimental.pallas import tpu as pltpu

_N = 512
_E = 4096
_CH = 16
_NN = _N * _N
_LN = _N * _CH
_ECHUNK = 1024
_JB = 2048
_NJ = _LN // _JB
_KB = 128
_NK = _LN // _KB


def _sel(rows, cols, mod):
    return (jax.lax.broadcasted_iota(jnp.int32, (rows, cols), 0) % mod
            == jax.lax.broadcasted_iota(jnp.int32, (rows, cols), 1) % mod
            ).astype(jnp.float32)


def _body(epsp_ref, ei_ref, d1_ref, c0_ref, c1_ref, w2_ref, vecs_ref,
          w_hbm, o_ref, a_ref, ab_ref, y_ref, h_ref, r_ref, sem):
    cp = pltpu.make_async_copy(w_hbm, h_ref, sem)
    cp.start()

    for t in range(_E // _ECHUNK):
        src = ei_ref[0, pl.ds(t * _ECHUNK, _ECHUNK)]
        dst = ei_ref[1, pl.ds(t * _ECHUNK, _ECHUNK)]
        rows = jax.lax.broadcasted_iota(jnp.int32, (_N, _ECHUNK), 0)
        u = (rows == dst[None, :]).astype(jnp.bfloat16)
        cols = jax.lax.broadcasted_iota(jnp.int32, (_ECHUNK, _N), 1)
        v = (cols == src[:, None]).astype(jnp.bfloat16)
        uv = jnp.dot(u, v, preferred_element_type=jnp.float32)
        if t == 0:
            a_ref[...] = uv
        else:
            a_ref[...] += uv
    ab_ref[...] = a_ref[...].astype(jnp.bfloat16)

    cp.wait()

    def y0_step(k, _):
        js = pl.ds(k * _KB, _KB)
        y_ref[:, js] = jnp.dot(h_ref[:, js], d1_ref[...],
                               preferred_element_type=jnp.float32
                               ).astype(jnp.bfloat16)
        return 0

    jax.lax.fori_loop(0, _NK, y0_step, 0)

    for layer in range(3):
        k_eps = epsp_ref[layer]

        def prop_step(j, carry):
            s1r, s2r = carry
            js = pl.ds(j * _JB, _JB)
            yb = y_ref[:, js]
            hb = (jnp.dot(ab_ref[...], yb, preferred_element_type=jnp.float32)
                  + k_eps * yb.astype(jnp.float32))
            h_ref[:, js] = hb
            s1r = s1r + jnp.sum(hb, axis=0, keepdims=True)
            s2r = s2r + jnp.sum(hb * hb, axis=0, keepdims=True)
            return s1r, s2r

        zrow = jnp.zeros((1, _JB), jnp.float32)
        s1r, s2r = jax.lax.fori_loop(0, _NJ, prop_step, (zrow, zrow))
        p = _sel(_JB, _CH, _CH)
        s1 = jnp.dot(s1r, p, preferred_element_type=jnp.float32)
        s2 = jnp.dot(s2r, p, preferred_element_type=jnp.float32)

        mu = s1 / _NN
        var = s2 / _NN - mu * mu
        g = vecs_ref[layer, 0:1, :]
        bt = vecs_ref[layer, 1:2, :]
        av = jax.lax.rsqrt(var + 1e-5) * g
        dv = bt - mu * av
        u = _sel(_CH, _KB, _CH)
        a128 = jnp.dot(av, u, preferred_element_type=jnp.float32)
        d128 = jnp.dot(dv, u, preferred_element_type=jnp.float32)

        if layer < 2:
            c_ref = c0_ref if layer == 0 else c1_ref
            bias = vecs_ref[layer, 2:3, :]
            b128 = jnp.dot(bias, u, preferred_element_type=jnp.float32)

            def fin_step(k, _):
                js = pl.ds(k * _KB, _KB)
                x = jnp.maximum(h_ref[:, js] * a128 + d128, 0.0)
                y_ref[:, js] = (jnp.dot(
                    x, c_ref[...], preferred_element_type=jnp.float32) + b128
                    ).astype(jnp.bfloat16)
                return 0

            jax.lax.fori_loop(0, _NK, fin_step, 0)
        else:
            pf = _sel(_KB, _CH, _CH)

            def red_step(k, _):
                js = pl.ds(k * _KB, _KB)
                x = jnp.maximum(h_ref[:, js] * a128 + d128, 0.0)
                rk = jnp.dot(x, pf, preferred_element_type=jnp.float32)
                r_ref[...] = jnp.where(k == 0, rk, r_ref[...] + rk)
                return 0

            jax.lax.fori_loop(0, _NK, red_step, 0)

            b2 = vecs_ref[layer, 2:3, :]
            o_ref[...] = (jnp.dot(r_ref[...], w2_ref[...],
                                  preferred_element_type=jnp.float32)
                          + _N * b2)


def kernel(W, edge_index, eps_0, W1_0, b1_0, g_0, bt_0, W2_0, b2_0,
           eps_1, W1_1, b1_1, g_1, bt_1, W2_1, b2_1,
           eps_2, W1_2, b1_2, g_2, bt_2, W2_2, b2_2):
    eye8 = jnp.eye(_KB // _CH, dtype=jnp.float32)
    d1 = jnp.kron(eye8, W1_0)
    c0 = jnp.kron(eye8, W2_0 @ W1_1)
    c1 = jnp.kron(eye8, W2_1 @ W1_2)
    pad = jnp.zeros((5, _CH), jnp.float32)
    vecs = jnp.stack([
        jnp.concatenate([g_0[None], bt_0[None], (b2_0 @ W1_1)[None], pad]),
        jnp.concatenate([g_1[None], bt_1[None], (b2_1 @ W1_2)[None], pad]),
        jnp.concatenate([g_2[None], bt_2[None], b2_2[None], pad]),
    ])
    epsp = 1.0 + jnp.concatenate([eps_0, eps_1, eps_2])

    return pl.pallas_call(
        _body,
        in_specs=[pl.BlockSpec(memory_space=pltpu.SMEM),
                  pl.BlockSpec(memory_space=pltpu.VMEM),
                  pl.BlockSpec(memory_space=pltpu.VMEM),
                  pl.BlockSpec(memory_space=pltpu.VMEM),
                  pl.BlockSpec(memory_space=pltpu.VMEM),
                  pl.BlockSpec(memory_space=pltpu.VMEM),
                  pl.BlockSpec(memory_space=pltpu.VMEM),
                  pl.BlockSpec(memory_space=pl.ANY)],
        out_specs=pl.BlockSpec(memory_space=pltpu.VMEM),
        out_shape=jax.ShapeDtypeStruct((_N, _CH), jnp.float32),
        scratch_shapes=[pltpu.VMEM((_N, _N), jnp.float32),
                        pltpu.VMEM((_N, _N), jnp.bfloat16),
                        pltpu.VMEM((_N, _LN), jnp.bfloat16),
                        pltpu.VMEM((_N, _LN), jnp.float32),
                        pltpu.VMEM((_N, _CH), jnp.float32),
                        pltpu.SemaphoreType.DMA],
    )(epsp, edge_index, d1, c0, c1, W2_2, vecs, W.reshape(_N, _LN))
